# Initial kernel scaffold; baseline (speedup 1.0000x reference)
import jax
import jax.numpy as jnp
from jax.experimental import pallas as pl


def kernel(x, edge_index, edge_weight, month_idx, dc_wz, dc_bz, dc_wr, dc_br, dc_wh, dc_bh, gru_wxz, gru_bxz, gru_whz, gru_bhz, gru_wxr, gru_bxr, gru_whr, gru_bhr, gru_wxh, gru_bxh, gru_whh, gru_bhh, bn1_g, bn1_b, bn1_m, bn1_v, bn2_g, bn2_b, bn2_m, bn2_v, lin1_w, lin1_b, lin2_w, lin2_b, lin3_w, lin3_b, emb, seas_w, seas_b):
    raise NotImplementedError("write your pallas kernel here")


# speedup vs baseline: 13.2739x; 13.2739x over previous
import functools

import jax
import jax.numpy as jnp
from jax import lax
from jax.experimental import pallas as pl
from jax.experimental.pallas import tpu as pltpu
from jax.experimental.pallas import tpu_sc as plsc

NC = 2
NS = 16
CH = 128


@functools.partial(jax.jit, static_argnums=(2, 3))
def _sc_degree(rowp, wp, n, n_chunks):
    mesh = plsc.VectorSubcoreMesh(core_axis_name="c", subcore_axis_name="s")
    nz = n // 5

    @functools.partial(
        pl.kernel,
        out_type=jax.ShapeDtypeStruct((NC, n), jnp.float32),
        mesh=mesh,
        scratch_types=[
            pltpu.VMEM((CH,), jnp.int32),
            pltpu.VMEM((CH,), jnp.float32),
            pltpu.VMEM((nz,), jnp.float32),
            pltpu.VMEM_SHARED((n,), jnp.float32),
        ],
        compiler_params=pltpu.CompilerParams(use_tc_tiling_on_sc=False),
    )
    def k(row_hbm, w_hbm, out_hbm, idxv, wv, zbuf, acc):
        c = lax.axis_index("c")
        s = lax.axis_index("s")
        wid = c * NS + s

        def zb(i, carry):
            zbuf[pl.ds(i * 16, 16)] = jnp.zeros((16,), jnp.float32)
            return carry

        lax.fori_loop(0, nz // 16, zb, 0)

        @pl.when(s < 5)
        def _():
            pltpu.sync_copy(zbuf, acc.at[pl.ds(s * nz, nz)])

        plsc.subcore_barrier()

        def body(g, carry):
            base = (wid * n_chunks + g) * CH
            pltpu.sync_copy(row_hbm.at[pl.ds(base, CH)], idxv)
            pltpu.sync_copy(w_hbm.at[pl.ds(base, CH)], wv)
            pltpu.sync_copy(wv, acc.at[idxv], add=True)
            return carry

        lax.fori_loop(0, n_chunks, body, 0)
        plsc.subcore_barrier()

        @pl.when(s < 5)
        def _():
            pltpu.sync_copy(acc.at[pl.ds(s * nz, nz)],
                            out_hbm.at[c, pl.ds(s * nz, nz)])

    return k(rowp, wp)


@functools.partial(jax.jit, static_argnums=(4, 5))
def _sc_scatter(rowp, colp, wp, hs, n, n_chunks):
    mesh = plsc.VectorSubcoreMesh(core_axis_name="c", subcore_axis_name="s")
    hid = hs.shape[1]
    nvr = hid // 16
    zr = n // (NS * 5)
    wr = n // NS

    @functools.partial(
        pl.kernel,
        out_type=jax.ShapeDtypeStruct((NC, n, hid), jnp.float32),
        mesh=mesh,
        scratch_types=[
            pltpu.VMEM((CH,), jnp.int32),
            pltpu.VMEM((CH,), jnp.int32),
            pltpu.VMEM((CH,), jnp.float32),
            pltpu.VMEM((CH, hid), jnp.float32),
            pltpu.VMEM((zr, hid), jnp.float32),
            pltpu.VMEM_SHARED((n, hid), jnp.float32),
            pltpu.SemaphoreType.DMA,
        ],
        compiler_params=pltpu.CompilerParams(use_tc_tiling_on_sc=False),
    )
    def k(row_hbm, col_hbm, w_hbm, hs_hbm, out_hbm,
          rowv, colv, wv, rows, zbuf, acc, sem):
        c = lax.axis_index("c")
        s = lax.axis_index("s")
        wid = c * NS + s

        def zb(i, carry):
            for j in range(nvr):
                zbuf[i, pl.ds(j * 16, 16)] = jnp.zeros((16,), jnp.float32)
            return carry

        lax.fori_loop(0, zr, zb, 0)
        for t in range(5):
            pltpu.sync_copy(zbuf, acc.at[pl.ds((s * 5 + t) * zr, zr)])
        plsc.subcore_barrier()

        def body(g, carry):
            base = (wid * n_chunks + g) * CH
            pltpu.sync_copy(row_hbm.at[pl.ds(base, CH)], rowv)
            pltpu.sync_copy(col_hbm.at[pl.ds(base, CH)], colv)
            pltpu.sync_copy(w_hbm.at[pl.ds(base, CH)], wv)
            pltpu.async_copy(hs_hbm.at[rowv], rows, sem).wait()

            def e_body(i, cc):
                w16 = wv[pl.ds(i * 16, 16)]
                for l in range(16):
                    e = i * 16 + l
                    wb = jnp.full((16,), w16[l], jnp.float32)
                    for j in range(nvr):
                        rows[e, pl.ds(j * 16, 16)] = (
                            rows[e, pl.ds(j * 16, 16)] * wb)
                return cc

            lax.fori_loop(0, CH // 16, e_body, 0)
            pltpu.sync_copy(rows, acc.at[colv], add=True)
            return carry

        lax.fori_loop(0, n_chunks, body, 0)
        plsc.subcore_barrier()
        pltpu.sync_copy(acc.at[pl.ds(s * wr, wr)],
                        out_hbm.at[c, pl.ds(s * wr, wr)])

    return k(rowp, colp, wp, hs)


def _elu(v):
    return jnp.where(v > 0, v, jnp.exp(v) - 1.0)


def _dinv_of(degp):
    deg = degp[:, 0] + degp[:, 1]
    return jnp.where(deg > 0, lax.rsqrt(jnp.where(deg > 0, deg, 1.0)), 0.0)


def _stage1_body(x_ref, degp_ref, wz_ref, wh_ref, bz_ref, bh_ref,
                 sc1_ref, sh1_ref, h_ref, hs_ref):
    xb = x_ref[...]
    z = jax.nn.sigmoid(
        jnp.dot(xb, wz_ref[...], preferred_element_type=jnp.float32)
        + bz_ref[...])
    ht = jnp.tanh(
        jnp.dot(xb, wh_ref[...], preferred_element_type=jnp.float32)
        + bh_ref[...])
    h = _elu((1.0 - z) * ht)
    h = h * sc1_ref[...] + sh1_ref[...]
    dinv = _dinv_of(degp_ref)
    h_ref[...] = h
    hs_ref[...] = h * dinv[:, None]


def _tc_stage1(x, degp, wz, wh, bz, bh, sc1, sh1, block):
    n, fin = x.shape
    hid = wz.shape[1]
    grid = (n // block,)
    full = lambda shape: pl.BlockSpec(shape, lambda i: tuple(0 for _ in shape))
    return pl.pallas_call(
        _stage1_body,
        grid=grid,
        in_specs=[
            pl.BlockSpec((block, fin), lambda i: (i, 0)),
            pl.BlockSpec((block, NC), lambda i: (i, 0)),
            full((fin, hid)), full((fin, hid)),
            full((1, hid)), full((1, hid)), full((1, hid)), full((1, hid)),
        ],
        out_specs=[
            pl.BlockSpec((block, hid), lambda i: (i, 0)),
            pl.BlockSpec((block, hid), lambda i: (i, 0)),
        ],
        out_shape=[
            jax.ShapeDtypeStruct((n, hid), jnp.float32),
            jax.ShapeDtypeStruct((n, hid), jnp.float32),
        ],
    )(x, degp, wz, wh, bz, bh, sc1, sh1)


def _stage3_body(parts_ref, degp_ref, h_ref,
                 wxz0_ref, wxz1_ref, wxh0_ref, wxh1_ref, bzg_ref, bhg_ref,
                 l1w_ref, l1b_ref, sc2_ref, sh2_ref,
                 l2w_ref, l2b_ref, w3_ref, b3_ref,
                 emb_ref, sw_ref, sb_ref, mon_ref, out_ref):
    sac = parts_ref[0] + parts_ref[1]
    dinv = _dinv_of(degp_ref)
    t1 = -(dinv[:, None]) * sac
    hb = h_ref[...]
    dot = lambda a, b: jnp.dot(a, b, preferred_element_type=jnp.float32)
    zg = jax.nn.sigmoid(dot(hb, wxz0_ref[...]) + dot(t1, wxz1_ref[...])
                        + bzg_ref[...])
    htg = jnp.tanh(dot(hb, wxh0_ref[...]) + dot(t1, wxh1_ref[...])
                   + bhg_ref[...])
    h2 = _elu((1.0 - zg) * htg)
    h3 = _elu(dot(h2, l1w_ref[...]) + l1b_ref[...])
    h3 = h3 * sc2_ref[...] + sh2_ref[...]
    h4 = _elu(dot(h3, l2w_ref[...]) + l2b_ref[...])
    month = mon_ref[0]
    me = emb_ref[pl.ds(month, 1), :]
    seas = dot(me, sw_ref[...]) + sb_ref[...]
    h4 = h4 + seas
    out_ref[...] = jnp.sum(h4 * w3_ref[...], axis=1, keepdims=True) + b3_ref[...]


def _tc_stage3(parts, degp, h, wxz0, wxz1, wxh0, wxh1, bzg, bhg,
               l1w, l1b, sc2, sh2, l2w, l2b, w3, b3,
               emb, sw, sb, month_idx, block):
    n, hid = h.shape
    h2d = l1w.shape[1]
    h4d = l2w.shape[1]
    grid = (n // block,)
    full = lambda shape: pl.BlockSpec(shape, lambda i: tuple(0 for _ in shape))
    return pl.pallas_call(
        _stage3_body,
        grid=grid,
        in_specs=[
            pl.BlockSpec((NC, block, hid), lambda i: (0, i, 0)),
            pl.BlockSpec((block, NC), lambda i: (i, 0)),
            pl.BlockSpec((block, hid), lambda i: (i, 0)),
            full((hid, hid)), full((hid, hid)),
            full((hid, hid)), full((hid, hid)),
            full((1, hid)), full((1, hid)),
            full((hid, h2d)), full((1, h2d)), full((1, h2d)), full((1, h2d)),
            full((h2d, h4d)), full((1, h4d)),
            full((1, h4d)), full((1, 1)),
            full(emb.shape), full(sw.shape), full((1, h4d)),
            pl.BlockSpec(memory_space=pltpu.SMEM),
        ],
        out_specs=pl.BlockSpec((block, 1), lambda i: (i, 0)),
        out_shape=jax.ShapeDtypeStruct((n, 1), jnp.float32),
    )(parts, degp, h, wxz0, wxz1, wxh0, wxh1, bzg, bhg,
      l1w, l1b, sc2, sh2, l2w, l2b, w3, b3, emb, sw, sb, month_idx)


def kernel(x, edge_index, edge_weight, month_idx,
           dc_wz, dc_bz, dc_wr, dc_br, dc_wh, dc_bh,
           gru_wxz, gru_bxz, gru_whz, gru_bhz,
           gru_wxr, gru_bxr, gru_whr, gru_bhr,
           gru_wxh, gru_bxh, gru_whh, gru_bhh,
           bn1_g, bn1_b, bn1_m, bn1_v,
           bn2_g, bn2_b, bn2_m, bn2_v,
           lin1_w, lin1_b, lin2_w, lin2_b, lin3_w, lin3_b,
           emb, seas_w, seas_b):
    n, fin = x.shape
    hid = dc_bz.shape[0]
    e = edge_index.shape[1]

    n_chunks = -(-e // (NC * NS * CH))
    epad = NC * NS * CH * n_chunks
    pad = epad - e
    row = edge_index[0]
    col = edge_index[1]
    if pad:
        zi = jnp.zeros((pad,), jnp.int32)
        row = jnp.concatenate([row, zi])
        col = jnp.concatenate([col, zi])
        wp = jnp.concatenate([edge_weight, jnp.zeros((pad,), jnp.float32)])
    else:
        wp = edge_weight

    degp = _sc_degree(row, wp, n, n_chunks).T

    wz = (dc_wz[0, 0] + dc_wz[1, 0])[:fin]
    wh = (dc_wh[0, 0] + dc_wh[1, 0])[:fin]
    sc1 = (bn1_g * lax.rsqrt(bn1_v + 1e-5)).reshape(1, hid)
    sh1 = (bn1_b - bn1_m * sc1[0]).reshape(1, hid)
    block = 1000
    h, hs = _tc_stage1(x, degp, wz, wh,
                       dc_bz.reshape(1, hid), dc_bh.reshape(1, hid),
                       sc1, sh1, block)

    parts = _sc_scatter(row, col, wp, hs, n, n_chunks)

    h2d = lin1_w.shape[1]
    h4d = lin2_w.shape[1]
    sc2 = (bn2_g * lax.rsqrt(bn2_v + 1e-5)).reshape(1, h2d)
    sh2 = (bn2_b - bn2_m * sc2[0]).reshape(1, h2d)
    out = _tc_stage3(
        parts, degp, h,
        gru_wxz[0], gru_wxz[1], gru_wxh[0], gru_wxh[1],
        (gru_bxz + gru_bhz).reshape(1, hid), (gru_bxh + gru_bhh).reshape(1, hid),
        lin1_w, lin1_b.reshape(1, h2d), sc2, sh2,
        lin2_w, lin2_b.reshape(1, h4d),
        lin3_w.reshape(1, h4d), lin3_b.reshape(1, 1),
        emb, seas_w, seas_b.reshape(1, h4d), month_idx, block)
    return out
